# baseline (speedup 1.0000x reference)
import functools

import jax
import jax.numpy as jnp
from jax import lax
from jax.experimental import pallas as pl
from jax.experimental.pallas import tpu as pltpu
from jax.experimental.pallas import tpu_sc as plsc

BW = 128
LANES = 16


@functools.lru_cache(maxsize=None)
def _make(batch: int, seq: int, vocab: int, embed: int):
    info = plsc.get_sparse_core_info()
    nc, ns = info.num_cores, info.num_subcores
    nw = nc * ns
    assert batch == nw * BW
    mesh = plsc.VectorSubcoreMesh(core_axis_name="c", subcore_axis_name="s")

    @functools.partial(
        pl.kernel,
        mesh=mesh,
        out_type=jax.ShapeDtypeStruct((seq, embed, batch), jnp.float32),
        scratch_types=[
            pltpu.VMEM((seq, BW), jnp.int32),
        ]
        + [pltpu.VMEM((BW, embed), jnp.float32) for _ in range(2)]
        + [pltpu.VMEM((embed, BW), jnp.float32) for _ in range(2)]
        + [pltpu.SemaphoreType.DMA for _ in range(2)]
        + [pltpu.SemaphoreType.DMA for _ in range(2)],
        compiler_params=pltpu.CompilerParams(
            use_tc_tiling_on_sc=False, needs_layout_passes=False
        ),
    )
    def body(xt_hbm, t_hbm, out_hbm, idx_v, *rest):
        gbuf = rest[0:2]
        tbuf = rest[2:4]
        gsem = rest[4:6]
        ssem = rest[6:8]
        wid = lax.axis_index("s") * nc + lax.axis_index("c")
        b0 = wid * BW
        pltpu.sync_copy(xt_hbm.at[:, pl.ds(b0, BW)], idx_v)

        def fire_gather(s, h):
            pltpu.async_copy(
                t_hbm.at[idx_v.at[s, pl.ds(0, BW)]], gbuf[h], gsem[h]
            )

        def wait_gather(h):
            pltpu.make_async_copy(
                t_hbm.at[pl.ds(0, BW), :], gbuf[h], gsem[h]
            ).wait()

        def fire_store(s, h):
            pltpu.async_copy(tbuf[h], out_hbm.at[s, :, pl.ds(b0, BW)], ssem[h])

        def wait_store(h):
            pltpu.make_async_copy(
                tbuf[h], out_hbm.at[0, :, pl.ds(0, BW)], ssem[h]
            ).wait()

        def transpose(h):
            src = gbuf[h]
            dst = tbuf[h]

            def r_body(t, carry):
                r0 = t * LANES
                rows = lax.iota(jnp.int32, LANES) + r0
                for e in range(embed):
                    col = jnp.full((LANES,), e, dtype=jnp.int32)
                    v = plsc.load_gather(src, [rows, col])
                    dst[e, pl.ds(r0, LANES)] = v
                return carry

            lax.fori_loop(0, BW // LANES, r_body, 0)

        fire_gather(0, 0)
        fire_gather(1, 1)

        def step(s, h, do_wait_store, do_fire_gather):
            wait_gather(h)
            if do_wait_store:
                wait_store(h)
            transpose(h)
            fire_store(s, h)
            if do_fire_gather:
                fire_gather(s + 2, h)

        step(0, 0, False, True)
        step(1, 1, False, True)

        def loop_body(t, carry):
            s = 2 * t
            step(s, 0, True, True)
            step(s + 1, 1, True, True)
            return carry

        n_pairs = seq // 2
        lax.fori_loop(1, n_pairs - 1, loop_body, 0)

        step(seq - 2, 0, True, False)
        step(seq - 1, 1, True, False)
        wait_store(0)
        wait_store(1)

    return body


def kernel(x, table):
    b, s = x.shape
    vocab, embed = table.shape
    xt = x.T.astype(jnp.int32)
    out_t = _make(b, s, vocab, embed)(xt, table)
    return out_t.transpose(2, 0, 1)


# speedup vs baseline: 1.3770x; 1.3770x over previous
import functools

import jax
import jax.numpy as jnp
from jax import lax
from jax.experimental import pallas as pl
from jax.experimental.pallas import tpu as pltpu
from jax.experimental.pallas import tpu_sc as plsc

NBUF = 4


@functools.lru_cache(maxsize=None)
def _make_gather(batch: int, seq: int, vocab: int, embed: int):
    info = plsc.get_sparse_core_info()
    nc, ns = info.num_cores, info.num_subcores
    nw = nc * ns
    assert batch % (nw * NBUF) == 0
    rows_w = batch // nw
    n_pass = rows_w // NBUF
    assert n_pass >= 2
    chunks = []
    off = 0
    while off < seq:
        w = min(128, seq - off)
        chunks.append((off, w))
        off += w
    assert all(o % 8 == 0 for o, _ in chunks)
    mesh = plsc.VectorSubcoreMesh(core_axis_name="c", subcore_axis_name="s")

    @functools.partial(
        pl.kernel,
        mesh=mesh,
        out_type=jax.ShapeDtypeStruct((batch, seq, embed), jnp.float32),
        scratch_types=[
            pltpu.VMEM((rows_w, seq), jnp.int32),
        ]
        + [pltpu.VMEM((seq, embed), jnp.float32) for _ in range(NBUF)]
        + [pltpu.SemaphoreType.DMA for _ in range(2 * NBUF)],
        compiler_params=pltpu.CompilerParams(use_tc_tiling_on_sc=False),
    )
    def gather_kernel(idx_hbm, table_hbm, out_hbm, idx_v, *rest):
        bufs = rest[:NBUF]
        gsem = rest[NBUF : 2 * NBUF]
        ssem = rest[2 * NBUF :]
        wid = lax.axis_index("s") * nc + lax.axis_index("c")
        row0 = wid * rows_w
        pltpu.sync_copy(idx_hbm.at[pl.ds(row0, rows_w), :], idx_v)

        def fire_gathers(r, b):
            for o, w in chunks:
                pltpu.async_copy(
                    table_hbm.at[idx_v.at[r, pl.ds(o, w)]],
                    bufs[b].at[pl.ds(o, w)],
                    gsem[b],
                )

        def wait_gathers(b):
            for o, w in chunks:
                pltpu.make_async_copy(
                    out_hbm.at[0, pl.ds(o, w), :],
                    bufs[b].at[pl.ds(o, w)],
                    gsem[b],
                ).wait()

        def fire_store(r, b):
            pltpu.async_copy(bufs[b], out_hbm.at[row0 + r], ssem[b])

        def wait_store(b):
            pltpu.make_async_copy(
                bufs[b], out_hbm.at[0], ssem[b]
            ).wait()

        def step(r, b, do_wait_store, do_fire_gather):
            wait_gathers(b)
            fire_store(r, b)
            if do_wait_store:
                wait_store((b + 2) % NBUF)
            if do_fire_gather:
                fire_gathers(r + 2, (b + 2) % NBUF)

        fire_gathers(0, 0)
        fire_gathers(1, 1)

        for b in range(NBUF):
            step(b, b, b >= 2, True)

        def body(t, carry):
            r_base = t * NBUF
            for b in range(NBUF):
                step(r_base + b, b, True, True)
            return carry

        lax.fori_loop(1, n_pass - 1, body, 0)

        r_base = (n_pass - 1) * NBUF
        for b in range(NBUF):
            r = r_base + b
            step(r, b, True, r + 2 < rows_w)

        wait_store((NBUF - 2) % NBUF)
        wait_store((NBUF - 1) % NBUF)

    return gather_kernel


def kernel(x, table):
    b, s = x.shape
    vocab, embed = table.shape
    xi = x.astype(jnp.int32)
    s0 = (s // 2 + 7) // 8 * 8
    out0 = _make_gather(b, s0, vocab, embed)(xi[:, :s0], table)
    out1 = _make_gather(b, s - s0, vocab, embed)(xi[:, s0:], table)
    return jnp.concatenate([out0, out1], axis=1)


# speedup vs baseline: 1.6494x; 1.1978x over previous
import functools

import jax
import jax.numpy as jnp
from jax import lax
from jax.experimental import pallas as pl
from jax.experimental.pallas import tpu as pltpu
from jax.experimental.pallas import tpu_sc as plsc

NBUF = 4


@functools.lru_cache(maxsize=None)
def _make_gather(batch: int, seq: int, vocab: int, embed: int):
    info = plsc.get_sparse_core_info()
    nc, ns = info.num_cores, info.num_subcores
    nw = nc * ns
    assert batch % (nw * NBUF) == 0
    rows_w = batch // nw
    n_pass = rows_w // NBUF
    assert n_pass >= 2
    chunks = []
    off = 0
    while off < seq:
        w = min(128, seq - off)
        chunks.append((off, w))
        off += w
    assert all(o % 8 == 0 for o, _ in chunks)
    mesh = plsc.VectorSubcoreMesh(core_axis_name="c", subcore_axis_name="s")

    @functools.partial(
        pl.kernel,
        mesh=mesh,
        out_type=jax.ShapeDtypeStruct((batch, seq, embed), jnp.float32),
        scratch_types=[
            pltpu.VMEM((rows_w, seq), jnp.int32),
        ]
        + [pltpu.VMEM((seq, embed), jnp.float32) for _ in range(NBUF)]
        + [pltpu.SemaphoreType.DMA for _ in range(2 * NBUF)],
        compiler_params=pltpu.CompilerParams(use_tc_tiling_on_sc=False),
    )
    def gather_kernel(idx_hbm, table_hbm, out_hbm, idx_v, *rest):
        bufs = rest[:NBUF]
        gsem = rest[NBUF : 2 * NBUF]
        ssem = rest[2 * NBUF :]
        wid = lax.axis_index("s") * nc + lax.axis_index("c")
        row0 = wid * rows_w
        pltpu.sync_copy(idx_hbm.at[pl.ds(row0, rows_w), :], idx_v)

        def fire_gathers(r, b):
            for o, w in chunks:
                pltpu.async_copy(
                    table_hbm.at[idx_v.at[r, pl.ds(o, w)]],
                    bufs[b].at[pl.ds(o, w)],
                    gsem[b],
                )

        def wait_gathers(b):
            for o, w in chunks:
                pltpu.make_async_copy(
                    out_hbm.at[0, pl.ds(o, w), :],
                    bufs[b].at[pl.ds(o, w)],
                    gsem[b],
                ).wait()

        def fire_store(r, b):
            pltpu.async_copy(bufs[b], out_hbm.at[row0 + r], ssem[b])

        def wait_store(b):
            pltpu.make_async_copy(
                bufs[b], out_hbm.at[0], ssem[b]
            ).wait()

        def step(r, b, do_wait_store, do_fire_gather):
            wait_gathers(b)
            fire_store(r, b)
            if do_wait_store:
                wait_store((b + 2) % NBUF)
            if do_fire_gather:
                fire_gathers(r + 2, (b + 2) % NBUF)

        fire_gathers(0, 0)
        fire_gathers(1, 1)

        for b in range(NBUF):
            step(b, b, b >= 2, True)

        def body(t, carry):
            r_base = t * NBUF
            for b in range(NBUF):
                step(r_base + b, b, True, True)
            return carry

        lax.fori_loop(1, n_pass - 1, body, 0)

        r_base = (n_pass - 1) * NBUF
        for b in range(NBUF):
            r = r_base + b
            step(r, b, True, r + 2 < rows_w)

        wait_store((NBUF - 2) % NBUF)
        wait_store((NBUF - 1) % NBUF)

    return gather_kernel


def kernel(x, table):
    b, s = x.shape
    vocab, embed = table.shape
    return _make_gather(b, s, vocab, embed)(x.astype(jnp.int32), table)
